# baseline (speedup 1.0000x reference)
import functools

import jax
import jax.numpy as jnp
from jax import lax
from jax.experimental import pallas as pl
from jax.experimental.pallas import tpu as pltpu
from jax.experimental.pallas import tpu_sc as plsc

NC = 2
NS = 16
NW = NC * NS
C = 256
DW = 8


NB = 4


@functools.cache
def _scatter_rows(F, E_pad, N_pad):
  chunks_per_tile = E_pad // (NW * C)
  rows_per_tile = N_pad // NS
  nr = chunks_per_tile // NB
  assert chunks_per_tile % NB == 0
  mesh = plsc.VectorSubcoreMesh(core_axis_name="c", subcore_axis_name="s",
                                num_cores=NC, num_subcores=NS)

  def body(hs, srcm, dstm, zeros, out, sidx, didx, acc, *bufs):
    rows = bufs[:NB]
    semg = bufs[NB:2 * NB]
    sems = bufs[2 * NB:]
    c = lax.axis_index("c")
    s = lax.axis_index("s")
    pltpu.sync_copy(zeros, acc.at[pl.ds(s * rows_per_tile, rows_per_tile)])
    first = (s * NC + c) * chunks_per_tile
    pltpu.sync_copy(srcm.at[pl.ds(first, chunks_per_tile)], sidx)
    pltpu.sync_copy(dstm.at[pl.ds(first, chunks_per_tile)], didx)
    plsc.subcore_barrier()

    for b in range(NB):
      pltpu.async_copy(hs.at[sidx.at[b]], rows[b], semg[b])

    def round_(g, carry):
      j0 = g * NB
      for b in range(NB):
        pltpu.make_async_copy(hs.at[sidx.at[j0 + b]], rows[b], semg[b]).wait()
        pltpu.async_copy(rows[b], acc.at[didx.at[j0 + b]], sems[b], add=True)

      @pl.when(g < nr - 1)
      def _refill():
        for b in range(NB):
          pltpu.make_async_copy(rows[b], acc.at[didx.at[0]], sems[b]).wait()
          pltpu.async_copy(hs.at[sidx.at[j0 + NB + b]], rows[b], semg[b])

      return carry

    lax.fori_loop(0, nr, round_, 0)
    for b in range(NB):
      pltpu.make_async_copy(rows[b], acc.at[didx.at[0]], sems[b]).wait()
    plsc.subcore_barrier()
    pltpu.sync_copy(acc.at[pl.ds(s * rows_per_tile, rows_per_tile)],
                    out.at[c, pl.ds(s * rows_per_tile, rows_per_tile)])

  return pl.kernel(
      body,
      out_type=jax.ShapeDtypeStruct((NC, N_pad, F), jnp.float32),
      mesh=mesh,
      scratch_types=(
          [pltpu.VMEM((E_pad // (NW * C), C), jnp.int32),
           pltpu.VMEM((E_pad // (NW * C), C), jnp.int32),
           pltpu.VMEM_SHARED((N_pad, F), jnp.float32)]
          + [pltpu.VMEM((C, F), jnp.float32) for _ in range(NB)]
          + [pltpu.SemaphoreType.DMA for _ in range(2 * NB)]
      ),
      compiler_params=pltpu.CompilerParams(use_tc_tiling_on_sc=False),
  )


@functools.cache
def _degrees(E_pad, N_pad):
  chunks_per_tile = E_pad // (NW * C)
  rows_per_tile = N_pad // NS
  mesh = plsc.VectorSubcoreMesh(core_axis_name="c", subcore_axis_name="s",
                                num_cores=NC, num_subcores=NS)

  nr = chunks_per_tile // NB
  assert chunks_per_tile % NB == 0

  def body(srcm, dstm, ones_h, zeros, out_o, out_i,
           sidx, didx, ones_v, dego, degi, *sems):
    semo = sems[:NB]
    semi = sems[NB:]
    c = lax.axis_index("c")
    s = lax.axis_index("s")
    pltpu.sync_copy(zeros, dego.at[pl.ds(s * rows_per_tile, rows_per_tile)])
    pltpu.sync_copy(zeros, degi.at[pl.ds(s * rows_per_tile, rows_per_tile)])
    pltpu.sync_copy(ones_h, ones_v)
    first = (s * NC + c) * chunks_per_tile
    pltpu.sync_copy(srcm.at[pl.ds(first, chunks_per_tile)], sidx)
    pltpu.sync_copy(dstm.at[pl.ds(first, chunks_per_tile)], didx)
    plsc.subcore_barrier()

    def round_(g, carry):
      j0 = g * NB
      for b in range(NB):
        @pl.when(g > 0)
        def _drain():
          pltpu.make_async_copy(ones_v, dego.at[sidx.at[0]], semo[b]).wait()
          pltpu.make_async_copy(ones_v, degi.at[didx.at[0]], semi[b]).wait()
        pltpu.async_copy(ones_v, dego.at[sidx.at[j0 + b]], semo[b], add=True)
        pltpu.async_copy(ones_v, degi.at[didx.at[j0 + b]], semi[b], add=True)
      return carry

    lax.fori_loop(0, nr, round_, 0)
    for b in range(NB):
      pltpu.make_async_copy(ones_v, dego.at[sidx.at[0]], semo[b]).wait()
      pltpu.make_async_copy(ones_v, degi.at[didx.at[0]], semi[b]).wait()
    plsc.subcore_barrier()
    sl = pl.ds(s * rows_per_tile, rows_per_tile)
    pltpu.sync_copy(dego.at[sl], out_o.at[c, sl])
    pltpu.sync_copy(degi.at[sl], out_i.at[c, sl])

  return pl.kernel(
      body,
      out_type=(jax.ShapeDtypeStruct((NC, N_pad, DW), jnp.float32),
                jax.ShapeDtypeStruct((NC, N_pad, DW), jnp.float32)),
      mesh=mesh,
      scratch_types=[
          pltpu.VMEM((E_pad // (NW * C), C), jnp.int32),
          pltpu.VMEM((E_pad // (NW * C), C), jnp.int32),
          pltpu.VMEM((C, DW), jnp.float32),
          pltpu.VMEM_SHARED((N_pad, DW), jnp.float32),
          pltpu.VMEM_SHARED((N_pad, DW), jnp.float32),
      ] + [pltpu.SemaphoreType.DMA for _ in range(2 * NB)],
      compiler_params=pltpu.CompilerParams(use_tc_tiling_on_sc=False),
  )


def _inv_sqrt(d0_ref, d1_ref):
  d = d0_ref[:, :1] + d1_ref[:, :1]
  return jnp.where(d > 0.0, 1.0 / jnp.sqrt(jnp.maximum(d, 1.0)), 0.0)


def _tc_first(R, N_pad, K, F):
  def body(x_ref, w_ref, o_ref):
    o_ref[:] = jnp.dot(x_ref[:], w_ref[:], preferred_element_type=jnp.float32)

  return pl.pallas_call(
      body,
      grid=(N_pad // R,),
      in_specs=[
          pl.BlockSpec((R, K), lambda i: (i, 0)),
          pl.BlockSpec((K, F), lambda i: (0, 0)),
      ],
      out_specs=pl.BlockSpec((R, F), lambda i: (i, 0)),
      out_shape=jax.ShapeDtypeStruct((N_pad, F), jnp.float32),
  )


def _tc_norm(R, N_pad, F):
  def body(do_ref, di_ref, t_ref, o_ref, s_ref):
    a = _inv_sqrt(do_ref[0], do_ref[1])
    b = _inv_sqrt(di_ref[0], di_ref[1])
    col = lax.broadcasted_iota(jnp.int32, (R, DW), 1)
    s_ref[:] = jnp.where(col == 0, a, jnp.where(col == 1, b, 0.0))
    o_ref[:] = t_ref[:] * a

  return pl.pallas_call(
      body,
      grid=(N_pad // R,),
      in_specs=[
          pl.BlockSpec((NC, R, DW), lambda i: (0, i, 0)),
          pl.BlockSpec((NC, R, DW), lambda i: (0, i, 0)),
          pl.BlockSpec((R, F), lambda i: (i, 0)),
      ],
      out_specs=(pl.BlockSpec((R, F), lambda i: (i, 0)),
                 pl.BlockSpec((R, DW), lambda i: (i, 0))),
      out_shape=(jax.ShapeDtypeStruct((N_pad, F), jnp.float32),
                 jax.ShapeDtypeStruct((N_pad, DW), jnp.float32)),
  )


def _tc_mid(R, N_pad, Fp, F):
  def body(p_ref, s_ref, bias_ref, w_ref, o_ref):
    h = jnp.maximum((p_ref[0] + p_ref[1]) * s_ref[:, 1:2] + bias_ref[0, 0],
                    0.0)
    o_ref[:] = jnp.dot(h, w_ref[:],
                       preferred_element_type=jnp.float32) * s_ref[:, :1]

  return pl.pallas_call(
      body,
      grid=(N_pad // R,),
      in_specs=[
          pl.BlockSpec((NC, R, Fp), lambda i: (0, i, 0)),
          pl.BlockSpec((R, DW), lambda i: (i, 0)),
          pl.BlockSpec((1, 1), lambda i: (0, 0)),
          pl.BlockSpec((Fp, F), lambda i: (0, 0)),
      ],
      out_specs=pl.BlockSpec((R, F), lambda i: (i, 0)),
      out_shape=jax.ShapeDtypeStruct((N_pad, F), jnp.float32),
  )


def _tc_last(R, N_pad, F):
  def body(p_ref, s_ref, bias_ref, o_ref):
    h = jnp.maximum((p_ref[0] + p_ref[1]) * s_ref[:, 1:2] + bias_ref[0, 0],
                    0.0)
    m = jnp.max(h, axis=1, keepdims=True)
    e = jnp.exp(h - m)
    o_ref[:] = e / jnp.sum(e, axis=1, keepdims=True)

  return pl.pallas_call(
      body,
      grid=(N_pad // R,),
      in_specs=[
          pl.BlockSpec((NC, R, F), lambda i: (0, i, 0)),
          pl.BlockSpec((R, DW), lambda i: (i, 0)),
          pl.BlockSpec((1, 1), lambda i: (0, 0)),
      ],
      out_specs=pl.BlockSpec((R, F), lambda i: (i, 0)),
      out_shape=jax.ShapeDtypeStruct((N_pad, F), jnp.float32),
  )


def kernel(x, edge_index, W1, W2, W3, W4, b1, b2, b3, b4):
  N, K = x.shape
  E = edge_index.shape[1]
  R = 1024
  N_pad = 10240
  E_pad = -(-E // (NW * C * 8)) * (NW * C * 8)
  assert N_pad >= N + 1 and N_pad % (NS * 8) == 0 and N_pad % R == 0

  pad = N + jnp.arange(E_pad - E, dtype=jnp.int32) % (N_pad - N)
  srcm = jnp.concatenate([edge_index[0], pad]).reshape(E_pad // C, C)
  dstm = jnp.concatenate([edge_index[1], pad]).reshape(E_pad // C, C)
  x_pad = jnp.pad(x, ((0, N_pad - N), (0, 0)))

  ones8 = jnp.ones((C, DW), jnp.float32)
  zeros8 = jnp.zeros((N_pad // NS, DW), jnp.float32)

  deg_o, deg_i = _degrees(E_pad, N_pad)(srcm, dstm, ones8, zeros8)

  dims = (W1.shape[1], W2.shape[1], W3.shape[1], W4.shape[1])

  t_raw = _tc_first(R, N_pad, K, dims[0])(x_pad, W1)
  t, scales = _tc_norm(R, N_pad, dims[0])(deg_o, deg_i, t_raw)
  for Fp, F, W, b in ((dims[0], dims[1], W2, b1),
                      (dims[1], dims[2], W3, b2),
                      (dims[2], dims[3], W4, b3)):
    p = _scatter_rows(Fp, E_pad, N_pad)(
        t, srcm, dstm, jnp.zeros((N_pad // NS, Fp), jnp.float32))
    t = _tc_mid(R, N_pad, Fp, F)(p, scales, b.reshape(1, 1), W)

  F4 = dims[3]
  p = _scatter_rows(F4, E_pad, N_pad)(
      t, srcm, dstm, jnp.zeros((N_pad // NS, F4), jnp.float32))
  out = _tc_last(R, N_pad, F4)(p, scales, b4.reshape(1, 1))
  return out[:N]


# speedup vs baseline: 1.0670x; 1.0670x over previous
import functools

import jax
import jax.numpy as jnp
from jax import lax
from jax.experimental import pallas as pl
from jax.experimental.pallas import tpu as pltpu
from jax.experimental.pallas import tpu_sc as plsc

NC = 2
NS = 16
NW = NC * NS
C = 128
DW = 8


NB = 8


@functools.cache
def _scatter_rows(F, E_pad, N_pad):
  chunks_per_tile = E_pad // (NW * C)
  rows_per_tile = N_pad // NS
  nr = chunks_per_tile // NB
  assert chunks_per_tile % NB == 0
  mesh = plsc.VectorSubcoreMesh(core_axis_name="c", subcore_axis_name="s",
                                num_cores=NC, num_subcores=NS)

  def body(hs, srcm, dstm, zeros, out, sidx, didx, acc, *bufs):
    rows = bufs[:NB]
    semg = bufs[NB:2 * NB]
    sems = bufs[2 * NB:]
    c = lax.axis_index("c")
    s = lax.axis_index("s")
    pltpu.sync_copy(zeros, acc.at[pl.ds(s * rows_per_tile, rows_per_tile)])
    first = (s * NC + c) * chunks_per_tile
    pltpu.sync_copy(srcm.at[pl.ds(first, chunks_per_tile)], sidx)
    pltpu.sync_copy(dstm.at[pl.ds(first, chunks_per_tile)], didx)
    plsc.subcore_barrier()

    for b in range(NB):
      pltpu.async_copy(hs.at[sidx.at[b]], rows[b], semg[b])

    def round_(g, carry):
      j0 = g * NB
      for b in range(NB):
        pltpu.make_async_copy(hs.at[sidx.at[j0 + b]], rows[b], semg[b]).wait()
        pltpu.async_copy(rows[b], acc.at[didx.at[j0 + b]], sems[b], add=True)

      @pl.when(g < nr - 1)
      def _refill():
        for b in range(NB):
          pltpu.make_async_copy(rows[b], acc.at[didx.at[0]], sems[b]).wait()
          pltpu.async_copy(hs.at[sidx.at[j0 + NB + b]], rows[b], semg[b])

      return carry

    lax.fori_loop(0, nr, round_, 0)
    for b in range(NB):
      pltpu.make_async_copy(rows[b], acc.at[didx.at[0]], sems[b]).wait()
    plsc.subcore_barrier()
    pltpu.sync_copy(acc.at[pl.ds(s * rows_per_tile, rows_per_tile)],
                    out.at[c, pl.ds(s * rows_per_tile, rows_per_tile)])

  return pl.kernel(
      body,
      out_type=jax.ShapeDtypeStruct((NC, N_pad, F), jnp.float32),
      mesh=mesh,
      scratch_types=(
          [pltpu.VMEM((E_pad // (NW * C), C), jnp.int32),
           pltpu.VMEM((E_pad // (NW * C), C), jnp.int32),
           pltpu.VMEM_SHARED((N_pad, F), jnp.float32)]
          + [pltpu.VMEM((C, F), jnp.float32) for _ in range(NB)]
          + [pltpu.SemaphoreType.DMA for _ in range(2 * NB)]
      ),
      compiler_params=pltpu.CompilerParams(use_tc_tiling_on_sc=False),
  )


@functools.cache
def _degrees(E_pad, N_pad):
  chunks_per_tile = E_pad // (NW * C)
  rows_per_tile = N_pad // NS
  mesh = plsc.VectorSubcoreMesh(core_axis_name="c", subcore_axis_name="s",
                                num_cores=NC, num_subcores=NS)

  nr = chunks_per_tile // NB
  assert chunks_per_tile % NB == 0

  def body(srcm, dstm, ones_h, zeros, out_o, out_i,
           sidx, didx, ones_v, dego, degi, *sems):
    semo = sems[:NB]
    semi = sems[NB:]
    c = lax.axis_index("c")
    s = lax.axis_index("s")
    pltpu.sync_copy(zeros, dego.at[pl.ds(s * rows_per_tile, rows_per_tile)])
    pltpu.sync_copy(zeros, degi.at[pl.ds(s * rows_per_tile, rows_per_tile)])
    pltpu.sync_copy(ones_h, ones_v)
    first = (s * NC + c) * chunks_per_tile
    pltpu.sync_copy(srcm.at[pl.ds(first, chunks_per_tile)], sidx)
    pltpu.sync_copy(dstm.at[pl.ds(first, chunks_per_tile)], didx)
    plsc.subcore_barrier()

    def round_(g, carry):
      j0 = g * NB
      for b in range(NB):
        @pl.when(g > 0)
        def _drain():
          pltpu.make_async_copy(ones_v, dego.at[sidx.at[0]], semo[b]).wait()
          pltpu.make_async_copy(ones_v, degi.at[didx.at[0]], semi[b]).wait()
        pltpu.async_copy(ones_v, dego.at[sidx.at[j0 + b]], semo[b], add=True)
        pltpu.async_copy(ones_v, degi.at[didx.at[j0 + b]], semi[b], add=True)
      return carry

    lax.fori_loop(0, nr, round_, 0)
    for b in range(NB):
      pltpu.make_async_copy(ones_v, dego.at[sidx.at[0]], semo[b]).wait()
      pltpu.make_async_copy(ones_v, degi.at[didx.at[0]], semi[b]).wait()
    plsc.subcore_barrier()
    sl = pl.ds(s * rows_per_tile, rows_per_tile)
    pltpu.sync_copy(dego.at[sl], out_o.at[c, sl])
    pltpu.sync_copy(degi.at[sl], out_i.at[c, sl])

  return pl.kernel(
      body,
      out_type=(jax.ShapeDtypeStruct((NC, N_pad, DW), jnp.float32),
                jax.ShapeDtypeStruct((NC, N_pad, DW), jnp.float32)),
      mesh=mesh,
      scratch_types=[
          pltpu.VMEM((E_pad // (NW * C), C), jnp.int32),
          pltpu.VMEM((E_pad // (NW * C), C), jnp.int32),
          pltpu.VMEM((C, DW), jnp.float32),
          pltpu.VMEM_SHARED((N_pad, DW), jnp.float32),
          pltpu.VMEM_SHARED((N_pad, DW), jnp.float32),
      ] + [pltpu.SemaphoreType.DMA for _ in range(2 * NB)],
      compiler_params=pltpu.CompilerParams(use_tc_tiling_on_sc=False),
  )


def _inv_sqrt(d0_ref, d1_ref):
  d = d0_ref[:, :1] + d1_ref[:, :1]
  return jnp.where(d > 0.0, 1.0 / jnp.sqrt(jnp.maximum(d, 1.0)), 0.0)


def _tc_first(R, N_pad, K, F):
  def body(x_ref, w_ref, o_ref):
    o_ref[:] = jnp.dot(x_ref[:], w_ref[:], preferred_element_type=jnp.float32)

  return pl.pallas_call(
      body,
      grid=(N_pad // R,),
      in_specs=[
          pl.BlockSpec((R, K), lambda i: (i, 0)),
          pl.BlockSpec((K, F), lambda i: (0, 0)),
      ],
      out_specs=pl.BlockSpec((R, F), lambda i: (i, 0)),
      out_shape=jax.ShapeDtypeStruct((N_pad, F), jnp.float32),
  )


def _tc_norm(R, N_pad, F):
  def body(do_ref, di_ref, t_ref, o_ref, s_ref):
    a = _inv_sqrt(do_ref[0], do_ref[1])
    b = _inv_sqrt(di_ref[0], di_ref[1])
    col = lax.broadcasted_iota(jnp.int32, (R, DW), 1)
    s_ref[:] = jnp.where(col == 0, a, jnp.where(col == 1, b, 0.0))
    o_ref[:] = t_ref[:] * a

  return pl.pallas_call(
      body,
      grid=(N_pad // R,),
      in_specs=[
          pl.BlockSpec((NC, R, DW), lambda i: (0, i, 0)),
          pl.BlockSpec((NC, R, DW), lambda i: (0, i, 0)),
          pl.BlockSpec((R, F), lambda i: (i, 0)),
      ],
      out_specs=(pl.BlockSpec((R, F), lambda i: (i, 0)),
                 pl.BlockSpec((R, DW), lambda i: (i, 0))),
      out_shape=(jax.ShapeDtypeStruct((N_pad, F), jnp.float32),
                 jax.ShapeDtypeStruct((N_pad, DW), jnp.float32)),
  )


def _unpack_h(p_ref, s_ref, bias, R, Fp):
  kp = 128 // Fp
  RP = R // kp
  q = p_ref[0] + p_ref[1]
  rown = lax.broadcasted_iota(jnp.int32, (R, RP), 0) // kp
  colm = lax.broadcasted_iota(jnp.int32, (R, RP), 1)
  g = (rown == colm).astype(jnp.float32)
  gq = jnp.dot(g, q, preferred_element_type=jnp.float32)
  b = s_ref[:, 1:2]
  hs = [jnp.maximum(gq[:, u * Fp:(u + 1) * Fp] * b + bias, 0.0)
        for u in range(kp)]
  return hs, kp


def _tc_mid(R, N_pad, Fp, F):
  kp = 128 // Fp

  def body(p_ref, s_ref, bias_ref, w_ref, o_ref):
    hs, _ = _unpack_h(p_ref, s_ref, bias_ref[0, 0], R, Fp)
    rmod = lax.broadcasted_iota(jnp.int32, (R, F), 0) % kp
    acc = jnp.zeros((R, F), jnp.float32)
    for u in range(kp):
      t_u = jnp.dot(hs[u], w_ref[:], preferred_element_type=jnp.float32)
      acc = acc + jnp.where(rmod == u, t_u, 0.0)
    o_ref[:] = acc * s_ref[:, :1]

  return pl.pallas_call(
      body,
      grid=(N_pad // R,),
      in_specs=[
          pl.BlockSpec((NC, R // kp, 128), lambda i: (0, i, 0)),
          pl.BlockSpec((R, DW), lambda i: (i, 0)),
          pl.BlockSpec((1, 1), lambda i: (0, 0)),
          pl.BlockSpec((Fp, F), lambda i: (0, 0)),
      ],
      out_specs=pl.BlockSpec((R, F), lambda i: (i, 0)),
      out_shape=jax.ShapeDtypeStruct((N_pad, F), jnp.float32),
  )


def _tc_last(R, N_pad, F):
  kp = 128 // F

  def body(p_ref, s_ref, bias_ref, o_ref):
    hs, _ = _unpack_h(p_ref, s_ref, bias_ref[0, 0], R, F)
    rmod = lax.broadcasted_iota(jnp.int32, (R, F), 0) % kp
    h = jnp.zeros((R, F), jnp.float32)
    for u in range(kp):
      h = h + jnp.where(rmod == u, hs[u], 0.0)
    m = jnp.max(h, axis=1, keepdims=True)
    e = jnp.exp(h - m)
    o_ref[:] = e / jnp.sum(e, axis=1, keepdims=True)

  return pl.pallas_call(
      body,
      grid=(N_pad // R,),
      in_specs=[
          pl.BlockSpec((NC, R // kp, 128), lambda i: (0, i, 0)),
          pl.BlockSpec((R, DW), lambda i: (i, 0)),
          pl.BlockSpec((1, 1), lambda i: (0, 0)),
      ],
      out_specs=pl.BlockSpec((R, F), lambda i: (i, 0)),
      out_shape=jax.ShapeDtypeStruct((N_pad, F), jnp.float32),
  )


def kernel(x, edge_index, W1, W2, W3, W4, b1, b2, b3, b4):
  N, K = x.shape
  E = edge_index.shape[1]
  R = 1024
  N_pad = 10240
  E_pad = -(-E // (NW * C * 8)) * (NW * C * 8)
  assert N_pad >= N + 1 and N_pad % (NS * 8) == 0 and N_pad % R == 0

  pad = N + jnp.arange(E_pad - E, dtype=jnp.int32) % (N_pad - N)
  srcm = jnp.concatenate([edge_index[0], pad]).reshape(E_pad // C, C)
  dstm = jnp.concatenate([edge_index[1], pad]).reshape(E_pad // C, C)
  x_pad = jnp.pad(x, ((0, N_pad - N), (0, 0)))

  ones8 = jnp.ones((C, DW), jnp.float32)
  zeros8 = jnp.zeros((N_pad // NS, DW), jnp.float32)

  deg_o, deg_i = _degrees(E_pad, N_pad)(srcm, dstm, ones8, zeros8)

  dims = (W1.shape[1], W2.shape[1], W3.shape[1], W4.shape[1])

  t_raw = _tc_first(R, N_pad, K, dims[0])(x_pad, W1)
  t, scales = _tc_norm(R, N_pad, dims[0])(deg_o, deg_i, t_raw)
  for Fp, F, W, b in ((dims[0], dims[1], W2, b1),
                      (dims[1], dims[2], W3, b2),
                      (dims[2], dims[3], W4, b3)):
    p = _scatter_rows(Fp, E_pad, N_pad)(
        t, srcm, dstm, jnp.zeros((N_pad // NS, Fp), jnp.float32))
    pk = p.reshape(NC, N_pad * Fp // 128, 128)
    t = _tc_mid(R, N_pad, Fp, F)(pk, scales, b.reshape(1, 1), W)

  F4 = dims[3]
  p = _scatter_rows(F4, E_pad, N_pad)(
      t, srcm, dstm, jnp.zeros((N_pad // NS, F4), jnp.float32))
  pk = p.reshape(NC, N_pad * F4 // 128, 128)
  out = _tc_last(R, N_pad, F4)(pk, scales, b4.reshape(1, 1))
  return out[:N]
